# baseline (speedup 1.0000x reference)
import functools

import jax
import jax.numpy as jnp
from jax import lax
from jax.experimental import pallas as pl
from jax.experimental.pallas import tpu as pltpu
from jax.experimental.pallas import tpu_sc as plsc

_D = 128
_CHUNK = 128
_NC, _NS = 2, 16
_NW = _NC * _NS


def _matmul2_block(x_ref, wd_ref, wg_ref, out_ref):
    h = jnp.maximum(
        jnp.dot(x_ref[...], wd_ref[...], preferred_element_type=jnp.float32), 0.0
    )
    out_ref[...] = jnp.dot(h, wg_ref[...], preferred_element_type=jnp.float32)


def _add_relu_block(a_ref, b_ref, out_ref):
    out_ref[...] = jnp.maximum(a_ref[...] + b_ref[...], 0.0)


@functools.partial(jax.jit, static_argnums=(2, 3))
def _sc_gather_scatter(args, zeros, G, NACC):
    hw, src2d, dst2d = args
    rows_init = NACC // _NS

    mesh = plsc.VectorSubcoreMesh(
        core_axis_name="c", subcore_axis_name="s", num_cores=_NC, num_subcores=_NS
    )

    @functools.partial(
        pl.kernel,
        out_type=jax.ShapeDtypeStruct((_NC, NACC, _D), jnp.float32),
        mesh=mesh,
        scratch_types=[
            pltpu.VMEM((1, _CHUNK), jnp.int32),
            pltpu.VMEM((1, _CHUNK), jnp.int32),
            pltpu.VMEM((_CHUNK, _D), jnp.float32),
            pltpu.VMEM_SHARED((NACC, _D), jnp.float32),
            pltpu.SemaphoreType.DMA,
        ],
    )
    def sc_kernel(hw_hbm, src_hbm, dst_hbm, zeros_hbm, out_hbm,
                  src_v, dst_v, rows_v, acc, sem_g):
        c = lax.axis_index("c")
        s = lax.axis_index("s")
        wid = s * _NC + c
        base = s * rows_init
        pltpu.sync_copy(zeros_hbm.at[pl.ds(base, rows_init)],
                        acc.at[pl.ds(base, rows_init)])
        plsc.subcore_barrier()

        @pl.loop(0, G)
        def _edge_chunk(g):
            t = wid * G + g
            pltpu.sync_copy(src_hbm.at[pl.ds(t, 1)], src_v)
            pltpu.sync_copy(dst_hbm.at[pl.ds(t, 1)], dst_v)
            pltpu.async_copy(hw_hbm.at[src_v.at[0]], rows_v, sem_g).wait()
            pltpu.sync_copy(rows_v, acc.at[dst_v.at[0]], add=True)

        plsc.subcore_barrier()
        pltpu.sync_copy(acc.at[pl.ds(base, rows_init)],
                        out_hbm.at[c, pl.ds(base, rows_init)])

    return sc_kernel(hw, src2d, dst2d, zeros)


def kernel(nodes_features, edge_index, W_dense, W_gcn1):
    N, D = nodes_features.shape
    E = edge_index.shape[1]
    BM = 1000

    hw = pl.pallas_call(
        _matmul2_block,
        grid=(N // BM,),
        in_specs=[
            pl.BlockSpec((BM, D), lambda i: (i, 0)),
            pl.BlockSpec((D, D), lambda i: (0, 0)),
            pl.BlockSpec((D, D), lambda i: (0, 0)),
        ],
        out_specs=pl.BlockSpec((BM, D), lambda i: (i, 0)),
        out_shape=jax.ShapeDtypeStruct((N, D), jnp.float32),
    )(nodes_features, W_dense, W_gcn1)

    G = -(-E // (_NW * _CHUNK))
    G = -(-G // 8) * 8
    EPAD = _NW * G * _CHUNK
    NACC = -(-(N + 1) // (_NS * 8)) * (_NS * 8)
    src = edge_index[0]
    dst = edge_index[1]
    pad = EPAD - E
    src2d = jnp.concatenate([src, jnp.zeros((pad,), jnp.int32)]).reshape(-1, _CHUNK)
    dst2d = jnp.concatenate([dst, jnp.full((pad,), N, jnp.int32)]).reshape(-1, _CHUNK)
    zeros = jnp.zeros((NACC, D), jnp.float32)

    partials = _sc_gather_scatter((hw, src2d, dst2d), zeros, G, NACC)

    p0 = partials[0, :N]
    p1 = partials[1, :N]
    out = pl.pallas_call(
        _add_relu_block,
        grid=(N // BM,),
        in_specs=[
            pl.BlockSpec((BM, D), lambda i: (i, 0)),
            pl.BlockSpec((BM, D), lambda i: (i, 0)),
        ],
        out_specs=pl.BlockSpec((BM, D), lambda i: (i, 0)),
        out_shape=jax.ShapeDtypeStruct((N, D), jnp.float32),
    )(p0, p1)
    return out


# speedup vs baseline: 2.1078x; 2.1078x over previous
import functools

import jax
import jax.numpy as jnp
from jax import lax
from jax.experimental import pallas as pl
from jax.experimental.pallas import tpu as pltpu
from jax.experimental.pallas import tpu_sc as plsc

_D = 128
_CHUNK = 128
_NC, _NS = 2, 16
_NW = _NC * _NS


def _matmul2_block(x_ref, wd_ref, wg_ref, out_ref):
    h = jnp.maximum(
        jnp.dot(x_ref[...], wd_ref[...], preferred_element_type=jnp.float32), 0.0
    )
    out_ref[...] = jnp.dot(h, wg_ref[...], preferred_element_type=jnp.float32)


def _add_relu_block(a_ref, b_ref, out_ref):
    out_ref[...] = jnp.maximum(a_ref[...] + b_ref[...], 0.0)


@functools.partial(jax.jit, static_argnums=(2, 3))
def _sc_gather_scatter(args, zeros, G, NACC):
    hw, src2d, dst2d = args
    rows_init = NACC // _NS

    mesh = plsc.VectorSubcoreMesh(
        core_axis_name="c", subcore_axis_name="s", num_cores=_NC, num_subcores=_NS
    )

    @functools.partial(
        pl.kernel,
        out_type=jax.ShapeDtypeStruct((_NC, NACC, _D), jnp.float32),
        mesh=mesh,
        scratch_types=[
            pltpu.VMEM((1, _CHUNK), jnp.int32),
            pltpu.VMEM((1, _CHUNK), jnp.int32),
            pltpu.VMEM((_CHUNK, _D), jnp.float32),
            pltpu.VMEM_SHARED((NACC, _D), jnp.float32),
            pltpu.SemaphoreType.DMA,
        ],
    )
    def sc_kernel(hw_hbm, src_hbm, dst_hbm, zeros_hbm, out_hbm,
                  src_v, dst_v, rows_v, acc, sem_g):
        c = lax.axis_index("c")
        s = lax.axis_index("s")
        wid = s * _NC + c
        base = s * rows_init
        pltpu.sync_copy(zeros_hbm.at[pl.ds(base, rows_init)],
                        acc.at[pl.ds(base, rows_init)])
        plsc.subcore_barrier()

        @pl.loop(0, G)
        def _edge_chunk(g):
            t = wid * G + g
            pltpu.sync_copy(src_hbm.at[pl.ds(t, 1)], src_v)
            pltpu.sync_copy(dst_hbm.at[pl.ds(t, 1)], dst_v)
            pltpu.async_copy(hw_hbm.at[src_v.at[0]], rows_v, sem_g).wait()
            pltpu.sync_copy(rows_v, acc.at[dst_v.at[0]], add=True)

        plsc.subcore_barrier()
        pltpu.sync_copy(acc.at[pl.ds(base, rows_init)],
                        out_hbm.at[c, pl.ds(base, rows_init)])

    return sc_kernel(hw, src2d, dst2d, zeros)


def kernel(nodes_features, edge_index, W_dense, W_gcn1):
    N, D = nodes_features.shape
    E = edge_index.shape[1]
    BM = 1000

    hw = pl.pallas_call(
        _matmul2_block,
        grid=(N // BM,),
        in_specs=[
            pl.BlockSpec((BM, D), lambda i: (i, 0)),
            pl.BlockSpec((D, D), lambda i: (0, 0)),
            pl.BlockSpec((D, D), lambda i: (0, 0)),
        ],
        out_specs=pl.BlockSpec((BM, D), lambda i: (i, 0)),
        out_shape=jax.ShapeDtypeStruct((N, D), jnp.float32),
    )(nodes_features, W_dense, W_gcn1)

    G = -(-E // (_NW * _CHUNK))
    EPAD = _NW * G * _CHUNK
    NACC = -(-(N + 1) // (_NS * 8)) * (_NS * 8)
    src = edge_index[0]
    dst = edge_index[1]
    pad = EPAD - E
    ar = jnp.arange(pad, dtype=jnp.int32)
    pad_src = (ar * 61) % N
    pad_dst = N + ar % (NACC - N)
    src2d = jnp.concatenate([src, pad_src]).reshape(-1, _CHUNK)
    dst2d = jnp.concatenate([dst, pad_dst]).reshape(-1, _CHUNK)
    zeros = jnp.zeros((NACC, D), jnp.float32)

    partials = _sc_gather_scatter((hw, src2d, dst2d), zeros, G, NACC)

    p0 = partials[0, :N]
    p1 = partials[1, :N]
    out = pl.pallas_call(
        _add_relu_block,
        grid=(N // BM,),
        in_specs=[
            pl.BlockSpec((BM, D), lambda i: (i, 0)),
            pl.BlockSpec((BM, D), lambda i: (i, 0)),
        ],
        out_specs=pl.BlockSpec((BM, D), lambda i: (i, 0)),
        out_shape=jax.ShapeDtypeStruct((N, D), jnp.float32),
    )(p0, p1)
    return out
